# Initial kernel scaffold; baseline (speedup 1.0000x reference)
import jax
import jax.numpy as jnp
from jax.experimental import pallas as pl


def kernel(frag_list):
    raise NotImplementedError("write your pallas kernel here")


# speedup vs baseline: 25.8270x; 25.8270x over previous
import functools

import jax
import jax.numpy as jnp
from jax import lax
from jax.experimental import pallas as pl
from jax.experimental.pallas import tpu as pltpu
from jax.experimental.pallas import tpu_sc as plsc

K = 4
N = 262144
NV = 3
RES = 256
HW = RES * RES
KV = K * NV
_AXES = [(0, 1, 2), (2, 1, 0), (0, 2, 1)]

_R = 512
_SLAB = 64
_WROWS = 64

NC = 2
NS = 16


def _project_kernel(u_ref, v_ref, d_ref, idx_ref, val_ref):
    u = u_ref[0]
    v = v_ref[0]
    d = d_ref[0]
    u0i = jnp.clip(u.astype(jnp.int32), 0, RES - 2)
    v0i = jnp.clip(v.astype(jnp.int32), 0, RES - 2)
    wu1 = jnp.clip(u - u0i.astype(jnp.float32), 0.0, 1.0)
    wv1 = jnp.clip(v - v0i.astype(jnp.float32), 0.0, 1.0)
    wu0 = 1.0 - wu1
    wv0 = 1.0 - wv1
    c00 = v0i * RES + u0i
    idx_ref[0, 0] = c00
    idx_ref[0, 1] = c00 + 1
    idx_ref[0, 2] = c00 + RES
    idx_ref[0, 3] = c00 + (RES + 1)
    w00 = wv0 * wu0
    w01 = wv0 * wu1
    w10 = wv1 * wu0
    w11 = wv1 * wu1
    val_ref[0, 0] = w00
    val_ref[0, 1] = w01
    val_ref[0, 2] = w10
    val_ref[0, 3] = w11
    val_ref[1, 0] = d * w00
    val_ref[1, 1] = d * w01
    val_ref[1, 2] = d * w10
    val_ref[1, 3] = d * w11


def _run_project(u, v, d):
    grid = (KV, _R // _SLAB)
    in_spec = pl.BlockSpec((1, _SLAB, _R), lambda i, j: (i, j, 0))
    return pl.pallas_call(
        _project_kernel,
        grid=grid,
        in_specs=[in_spec, in_spec, in_spec],
        out_specs=[
            pl.BlockSpec((1, 4, _SLAB, _R), lambda i, j: (i, 0, j, 0)),
            pl.BlockSpec((2, 4, _SLAB, _R), lambda i, j: (i, 0, j, 0)),
        ],
        out_shape=[
            jax.ShapeDtypeStruct((KV, 4, _R, _R), jnp.int32),
            jax.ShapeDtypeStruct((2 * KV, 4, _R, _R), jnp.float32),
        ],
        compiler_params=pltpu.CompilerParams(
            dimension_semantics=("parallel", "parallel")),
    )(u, v, d)


_NTASK = 96
_TPT = 3
_WIN = 64
_ROWS_PER_TASK = N // 128
_GR = HW // 128


def _make_sc_scatter():
    mesh = plsc.VectorSubcoreMesh(core_axis_name="c", subcore_axis_name="s",
                                  num_cores=NC, num_subcores=NS)
    scratch = [
        pltpu.VMEM((_GR, 128), jnp.float32),
        pltpu.VMEM((_WIN, 128), jnp.int32),
        pltpu.VMEM((_WIN, 128), jnp.float32),
    ]

    @functools.partial(
        pl.kernel,
        mesh=mesh,
        out_type=jax.ShapeDtypeStruct((_NTASK * _GR, 128), jnp.float32),
        scratch_types=scratch,
        compiler_params=pltpu.CompilerParams(needs_layout_passes=False),
    )
    def sc_scatter(idx_hbm, val_hbm, zeros_hbm, out_hbm, grid_v, idx_b,
                   val_b):
        cid = lax.axis_index("c")
        sid = lax.axis_index("s")
        wid = sid * NC + cid
        @pl.loop(0, _TPT)
        def _task(t):
            task = wid * _TPT + t
            pair = task // 8
            arr = (task // 4) % 2
            q = task % 4
            idx_row0 = pair * (4 * N // 128) + q * _ROWS_PER_TASK
            val_row0 = (2 * pair + arr) * (4 * N // 128) + q * _ROWS_PER_TASK
            pltpu.sync_copy(zeros_hbm, grid_v)

            @pl.loop(0, _ROWS_PER_TASK // _WIN)
            def _win(w):
                r0 = w * _WIN
                pltpu.sync_copy(idx_hbm.at[pl.ds(idx_row0 + r0, _WIN)],
                                idx_b)
                pltpu.sync_copy(val_hbm.at[pl.ds(val_row0 + r0, _WIN)],
                                val_b)

                @pl.loop(0, _WIN)
                def _row(i):
                    for c in range(8):
                        sl = pl.ds(c * 16, 16)
                        idx = idx_b[i, sl]
                        plsc.addupdate_scatter(
                            grid_v, [idx >> 7, idx & 127], val_b[i, sl])

            pltpu.sync_copy(
                grid_v, out_hbm.at[pl.ds(task * _GR, _GR)])

    return sc_scatter


_make_sc_scatter = functools.cache(_make_sc_scatter)


def _finalize_kernel(cnt_ref, dw_ref, img_ref, cnt_out_ref):
    cnt = cnt_ref[0, 0].sum(axis=0)
    dw = dw_ref[0, 1].sum(axis=0)
    img_ref[0, 0] = dw / jnp.maximum(cnt, 1e-6)
    img_ref[0, 1] = (cnt > 0).astype(jnp.float32)
    cnt_out_ref[0] = cnt


def _run_finalize(acc):
    in_spec = pl.BlockSpec((1, 2, 4, _R, 128), lambda i: (i, 0, 0, 0, 0))
    return pl.pallas_call(
        _finalize_kernel,
        grid=(KV,),
        in_specs=[in_spec, in_spec],
        out_specs=[
            pl.BlockSpec((1, 2, _R, 128), lambda i: (i, 0, 0, 0)),
            pl.BlockSpec((1, _R, 128), lambda i: (i, 0, 0)),
        ],
        out_shape=[
            jax.ShapeDtypeStruct((KV, 2, _R, 128), jnp.float32),
            jax.ShapeDtypeStruct((KV, _R, 128), jnp.float32),
        ],
    )(acc, acc)


def kernel(frag_list):
    pts_n_list = []
    for k in range(K):
        pts = frag_list[k]
        centroid = pts.mean(axis=0, keepdims=True)
        pts = pts - centroid
        scale = jnp.maximum(jnp.max(jnp.abs(pts)), 1e-6)
        pts_n_list.append(pts / scale * 0.95)
    pts_n = jnp.stack(pts_n_list, axis=0)

    u_planes = []
    v_planes = []
    d_planes = []
    for v in range(NV):
        u_ax, v_ax, d_ax = _AXES[v]
        u_planes.append((pts_n[:, :, u_ax] + 0.95) / 1.9 * (RES - 1))
        v_planes.append((pts_n[:, :, v_ax] + 0.95) / 1.9 * (RES - 1))
        d_planes.append((pts_n[:, :, d_ax] + 0.95) / 1.9)
    u = jnp.stack(u_planes, axis=1).reshape(KV, _R, _R)
    v = jnp.stack(v_planes, axis=1).reshape(KV, _R, _R)
    d = jnp.stack(d_planes, axis=1).reshape(KV, _R, _R)

    scat_idx, scat_val = _run_project(u, v, d)

    acc_flat = _make_sc_scatter()(
        scat_idx.reshape(KV * 4 * N // 128, 128),
        scat_val.reshape(2 * KV * 4 * N // 128, 128),
        jnp.zeros((_GR, 128), jnp.float32),
    )
    acc = acc_flat.reshape(KV, 2, 4, _R, 128)

    images_flat, counts_flat = _run_finalize(acc)
    images = images_flat.reshape(K, NV, 2, RES, RES)
    counts = counts_flat.reshape(K, NV, HW)

    pix_corners = scat_idx.reshape(K, NV, 4, N).transpose(0, 3, 1, 2)
    pix_weights = (scat_val.reshape(KV, 2, 4, N)[:, 0]
                   .reshape(K, NV, 4, N).transpose(0, 3, 1, 2))
    return images, pix_corners, pix_weights, counts


# speedup vs baseline: 30.1080x; 1.1658x over previous
import functools

import jax
import jax.numpy as jnp
from jax import lax
from jax.experimental import pallas as pl
from jax.experimental.pallas import tpu as pltpu
from jax.experimental.pallas import tpu_sc as plsc

K = 4
N = 262144
NV = 3
RES = 256
HW = RES * RES
KV = K * NV
_AXES = [(0, 1, 2), (2, 1, 0), (0, 2, 1)]

_R = 512
_SLAB = 64
_WROWS = 64

NC = 2
NS = 16


def _project_kernel(u_ref, v_ref, d_ref, idx_ref, val_ref):
    u = u_ref[0]
    v = v_ref[0]
    d = d_ref[0]
    u0i = jnp.clip(u.astype(jnp.int32), 0, RES - 2)
    v0i = jnp.clip(v.astype(jnp.int32), 0, RES - 2)
    wu1 = jnp.clip(u - u0i.astype(jnp.float32), 0.0, 1.0)
    wv1 = jnp.clip(v - v0i.astype(jnp.float32), 0.0, 1.0)
    wu0 = 1.0 - wu1
    wv0 = 1.0 - wv1
    c00 = v0i * RES + u0i
    idx_ref[0, 0] = c00
    idx_ref[0, 1] = c00 + 1
    idx_ref[0, 2] = c00 + RES
    idx_ref[0, 3] = c00 + (RES + 1)
    w00 = wv0 * wu0
    w01 = wv0 * wu1
    w10 = wv1 * wu0
    w11 = wv1 * wu1
    val_ref[0, 0] = w00
    val_ref[0, 1] = w01
    val_ref[0, 2] = w10
    val_ref[0, 3] = w11
    val_ref[1, 0] = d * w00
    val_ref[1, 1] = d * w01
    val_ref[1, 2] = d * w10
    val_ref[1, 3] = d * w11


def _run_project(u, v, d):
    grid = (KV, _R // _SLAB)
    in_spec = pl.BlockSpec((1, _SLAB, _R), lambda i, j: (i, j, 0))
    return pl.pallas_call(
        _project_kernel,
        grid=grid,
        in_specs=[in_spec, in_spec, in_spec],
        out_specs=[
            pl.BlockSpec((1, 4, _SLAB, _R), lambda i, j: (i, 0, j, 0)),
            pl.BlockSpec((2, 4, _SLAB, _R), lambda i, j: (i, 0, j, 0)),
        ],
        out_shape=[
            jax.ShapeDtypeStruct((KV, 4, _R, _R), jnp.int32),
            jax.ShapeDtypeStruct((2 * KV, 4, _R, _R), jnp.float32),
        ],
        compiler_params=pltpu.CompilerParams(
            dimension_semantics=("parallel", "parallel")),
    )(u, v, d)


_NTASK = 96
_TPT = 3
_WIN = 64
_ROWS_PER_TASK = N // 128
_GR = HW // 128


def _make_sc_scatter():
    mesh = plsc.VectorSubcoreMesh(core_axis_name="c", subcore_axis_name="s",
                                  num_cores=NC, num_subcores=NS)
    scratch = [
        pltpu.VMEM((_GR, 128), jnp.float32),
        pltpu.VMEM((_WIN, 128), jnp.int32),
        pltpu.VMEM((_WIN, 128), jnp.float32),
    ]

    @functools.partial(
        pl.kernel,
        mesh=mesh,
        out_type=jax.ShapeDtypeStruct((_NTASK * _GR, 128), jnp.float32),
        scratch_types=scratch,
        compiler_params=pltpu.CompilerParams(needs_layout_passes=False),
    )
    def sc_scatter(idx_hbm, val_hbm, zeros_hbm, out_hbm, grid_v, idx_b,
                   val_b):
        cid = lax.axis_index("c")
        sid = lax.axis_index("s")
        wid = sid * NC + cid
        @pl.loop(0, _TPT)
        def _task(t):
            task = wid * _TPT + t
            pair = task // 8
            arr = (task // 4) % 2
            q = task % 4
            idx_row0 = pair * (4 * N // 128) + q * _ROWS_PER_TASK
            val_row0 = (2 * pair + arr) * (4 * N // 128) + q * _ROWS_PER_TASK
            pltpu.sync_copy(zeros_hbm, grid_v)

            @pl.loop(0, _ROWS_PER_TASK // _WIN)
            def _win(w):
                r0 = w * _WIN
                pltpu.sync_copy(idx_hbm.at[pl.ds(idx_row0 + r0, _WIN)],
                                idx_b)
                pltpu.sync_copy(val_hbm.at[pl.ds(val_row0 + r0, _WIN)],
                                val_b)

                @pl.loop(0, _WIN)
                def _row(i):
                    idxs = [idx_b[i, pl.ds(c * 16, 16)] for c in range(8)]
                    vals = [val_b[i, pl.ds(c * 16, 16)] for c in range(8)]
                    for c in range(8):
                        plsc.addupdate_scatter(
                            grid_v, [idxs[c] >> 7, idxs[c] & 127], vals[c])

            pltpu.sync_copy(
                grid_v, out_hbm.at[pl.ds(task * _GR, _GR)])

    return sc_scatter


_make_sc_scatter = functools.cache(_make_sc_scatter)


def _finalize_kernel(cnt_ref, dw_ref, img_ref, cnt_out_ref):
    cnt = cnt_ref[0, 0].sum(axis=0)
    dw = dw_ref[0, 1].sum(axis=0)
    img_ref[0, 0] = dw / jnp.maximum(cnt, 1e-6)
    img_ref[0, 1] = (cnt > 0).astype(jnp.float32)
    cnt_out_ref[0] = cnt


def _run_finalize(acc):
    in_spec = pl.BlockSpec((1, 2, 4, _R, 128), lambda i: (i, 0, 0, 0, 0))
    return pl.pallas_call(
        _finalize_kernel,
        grid=(KV,),
        in_specs=[in_spec, in_spec],
        out_specs=[
            pl.BlockSpec((1, 2, _R, 128), lambda i: (i, 0, 0, 0)),
            pl.BlockSpec((1, _R, 128), lambda i: (i, 0, 0)),
        ],
        out_shape=[
            jax.ShapeDtypeStruct((KV, 2, _R, 128), jnp.float32),
            jax.ShapeDtypeStruct((KV, _R, 128), jnp.float32),
        ],
    )(acc, acc)


def kernel(frag_list):
    pts_n_list = []
    for k in range(K):
        pts = frag_list[k]
        centroid = pts.mean(axis=0, keepdims=True)
        pts = pts - centroid
        scale = jnp.maximum(jnp.max(jnp.abs(pts)), 1e-6)
        pts_n_list.append(pts / scale * 0.95)
    pts_n = jnp.stack(pts_n_list, axis=0)

    u_planes = []
    v_planes = []
    d_planes = []
    for v in range(NV):
        u_ax, v_ax, d_ax = _AXES[v]
        u_planes.append((pts_n[:, :, u_ax] + 0.95) / 1.9 * (RES - 1))
        v_planes.append((pts_n[:, :, v_ax] + 0.95) / 1.9 * (RES - 1))
        d_planes.append((pts_n[:, :, d_ax] + 0.95) / 1.9)
    u = jnp.stack(u_planes, axis=1).reshape(KV, _R, _R)
    v = jnp.stack(v_planes, axis=1).reshape(KV, _R, _R)
    d = jnp.stack(d_planes, axis=1).reshape(KV, _R, _R)

    scat_idx, scat_val = _run_project(u, v, d)

    acc_flat = _make_sc_scatter()(
        scat_idx.reshape(KV * 4 * N // 128, 128),
        scat_val.reshape(2 * KV * 4 * N // 128, 128),
        jnp.zeros((_GR, 128), jnp.float32),
    )
    acc = acc_flat.reshape(KV, 2, 4, _R, 128)

    images_flat, counts_flat = _run_finalize(acc)
    images = images_flat.reshape(K, NV, 2, RES, RES)
    counts = counts_flat.reshape(K, NV, HW)

    pix_corners = scat_idx.reshape(K, NV, 4, N).transpose(0, 3, 1, 2)
    pix_weights = (scat_val.reshape(KV, 2, 4, N)[:, 0]
                   .reshape(K, NV, 4, N).transpose(0, 3, 1, 2))
    return images, pix_corners, pix_weights, counts


# speedup vs baseline: 54.6417x; 1.8149x over previous
import functools

import jax
import jax.numpy as jnp
from jax import lax
from jax.experimental import pallas as pl
from jax.experimental.pallas import tpu as pltpu
from jax.experimental.pallas import tpu_sc as plsc

K = 4
N = 262144
NV = 3
RES = 256
HW = RES * RES
KV = K * NV
_AXES = [(0, 1, 2), (2, 1, 0), (0, 2, 1)]

_R = 512
_SLAB = 64
_WROWS = 64

NC = 2
NS = 16


def _project_kernel(p_ref, idx_ref, val_ref):
    for vi in range(NV):
        u_ax, v_ax, d_ax = _AXES[vi]
        u = p_ref[0, u_ax] * float(RES - 1)
        v = p_ref[0, v_ax] * float(RES - 1)
        d = p_ref[0, d_ax]
        u0i = jnp.clip(u.astype(jnp.int32), 0, RES - 2)
        v0i = jnp.clip(v.astype(jnp.int32), 0, RES - 2)
        wu1 = jnp.clip(u - u0i.astype(jnp.float32), 0.0, 1.0)
        wv1 = jnp.clip(v - v0i.astype(jnp.float32), 0.0, 1.0)
        wu0 = 1.0 - wu1
        wv0 = 1.0 - wv1
        c00 = v0i * RES + u0i
        idx_ref[vi, 0] = c00
        idx_ref[vi, 1] = c00 + 1
        idx_ref[vi, 2] = c00 + RES
        idx_ref[vi, 3] = c00 + (RES + 1)
        w00 = wv0 * wu0
        w01 = wv0 * wu1
        w10 = wv1 * wu0
        w11 = wv1 * wu1
        val_ref[2 * vi, 0] = w00
        val_ref[2 * vi, 1] = w01
        val_ref[2 * vi, 2] = w10
        val_ref[2 * vi, 3] = w11
        val_ref[2 * vi + 1, 0] = d * w00
        val_ref[2 * vi + 1, 1] = d * w01
        val_ref[2 * vi + 1, 2] = d * w10
        val_ref[2 * vi + 1, 3] = d * w11


def _run_project(p):
    grid = (K, _R // _SLAB)
    return pl.pallas_call(
        _project_kernel,
        grid=grid,
        in_specs=[pl.BlockSpec((1, 3, _SLAB, _R), lambda i, j: (i, 0, j, 0))],
        out_specs=[
            pl.BlockSpec((NV, 4, _SLAB, _R), lambda i, j: (i, 0, j, 0)),
            pl.BlockSpec((2 * NV, 4, _SLAB, _R), lambda i, j: (i, 0, j, 0)),
        ],
        out_shape=[
            jax.ShapeDtypeStruct((KV, 4, _R, _R), jnp.int32),
            jax.ShapeDtypeStruct((2 * KV, 4, _R, _R), jnp.float32),
        ],
        compiler_params=pltpu.CompilerParams(
            dimension_semantics=("parallel", "parallel")),
    )(p)


_NTASK = 96
_TPT = 3
_WIN = 64
_ROWS_PER_TASK = N // 128
_GR = HW // 128


def _make_sc_scatter():
    mesh = plsc.VectorSubcoreMesh(core_axis_name="c", subcore_axis_name="s",
                                  num_cores=NC, num_subcores=NS)
    scratch = [
        pltpu.VMEM((_GR, 128), jnp.float32),
        pltpu.VMEM((_WIN, 128), jnp.int32),
        pltpu.VMEM((_WIN, 128), jnp.int32),
        pltpu.VMEM((_WIN, 128), jnp.float32),
        pltpu.VMEM((_WIN, 128), jnp.float32),
        pltpu.SemaphoreType.DMA,
        pltpu.SemaphoreType.DMA,
        pltpu.SemaphoreType.DMA,
    ]
    nwin = _ROWS_PER_TASK // _WIN

    @functools.partial(
        pl.kernel,
        mesh=mesh,
        out_type=jax.ShapeDtypeStruct((_NTASK * _GR, 128), jnp.float32),
        scratch_types=scratch,
        compiler_params=pltpu.CompilerParams(needs_layout_passes=False),
    )
    def sc_scatter(idx_hbm, val_hbm, zeros_hbm, out_hbm, grid_v, idx_b0,
                   idx_b1, val_b0, val_b1, sem0, sem1, semz):
        cid = lax.axis_index("c")
        sid = lax.axis_index("s")
        wid = sid * NC + cid
        idx_bufs = (idx_b0, idx_b1)
        val_bufs = (val_b0, val_b1)
        sems = (sem0, sem1)

        @pl.loop(0, _TPT)
        def _task(t):
            task = wid * _TPT + t
            pair = task // 8
            arr = (task // 4) % 2
            q = task % 4
            idx_row0 = pair * (4 * N // 128) + q * _ROWS_PER_TASK
            val_row0 = (2 * pair + arr) * (4 * N // 128) + q * _ROWS_PER_TASK
            pltpu.async_copy(zeros_hbm, grid_v, semz)
            pltpu.async_copy(idx_hbm.at[pl.ds(idx_row0, _WIN)], idx_b0,
                             sem0)
            pltpu.async_copy(val_hbm.at[pl.ds(val_row0, _WIN)], val_b0,
                             sem0)
            pltpu.make_async_copy(zeros_hbm, grid_v, semz).wait()

            @pl.loop(0, nwin // 2)
            def _w2(w2):
                for b in range(2):
                    w = 2 * w2 + b
                    nb = 1 - b

                    @pl.when(w < nwin - 1)
                    def _prefetch():
                        r0 = (w + 1) * _WIN
                        pltpu.async_copy(
                            idx_hbm.at[pl.ds(idx_row0 + r0, _WIN)],
                            idx_bufs[nb], sems[nb])
                        pltpu.async_copy(
                            val_hbm.at[pl.ds(val_row0 + r0, _WIN)],
                            val_bufs[nb], sems[nb])

                    pltpu.make_async_copy(
                        idx_hbm.at[pl.ds(idx_row0, _WIN)], idx_bufs[b],
                        sems[b]).wait()
                    pltpu.make_async_copy(
                        val_hbm.at[pl.ds(val_row0, _WIN)], val_bufs[b],
                        sems[b]).wait()

                    @pl.loop(0, _WIN)
                    def _row(i):
                        idxs = [idx_bufs[b][i, pl.ds(c * 16, 16)]
                                for c in range(8)]
                        vals = [val_bufs[b][i, pl.ds(c * 16, 16)]
                                for c in range(8)]
                        for c in range(8):
                            plsc.addupdate_scatter(
                                grid_v, [idxs[c] >> 7, idxs[c] & 127],
                                vals[c])

            pltpu.sync_copy(
                grid_v, out_hbm.at[pl.ds(task * _GR, _GR)])

    return sc_scatter


_make_sc_scatter = functools.cache(_make_sc_scatter)


def _finalize_kernel(cnt_ref, dw_ref, img_ref, cnt_out_ref):
    cnt = cnt_ref[0, 0].sum(axis=0)
    dw = dw_ref[0, 1].sum(axis=0)
    img_ref[0, 0] = dw / jnp.maximum(cnt, 1e-6)
    img_ref[0, 1] = (cnt > 0).astype(jnp.float32)
    cnt_out_ref[0] = cnt


def _run_finalize(acc):
    in_spec = pl.BlockSpec((1, 2, 4, _R, 128), lambda i: (i, 0, 0, 0, 0))
    return pl.pallas_call(
        _finalize_kernel,
        grid=(KV,),
        in_specs=[in_spec, in_spec],
        out_specs=[
            pl.BlockSpec((1, 2, _R, 128), lambda i: (i, 0, 0, 0)),
            pl.BlockSpec((1, _R, 128), lambda i: (i, 0, 0)),
        ],
        out_shape=[
            jax.ShapeDtypeStruct((KV, 2, _R, 128), jnp.float32),
            jax.ShapeDtypeStruct((KV, _R, 128), jnp.float32),
        ],
    )(acc, acc)


def kernel(frag_list):
    cents = []
    scales = []
    for k in range(K):
        pts = frag_list[k]
        centroid = pts.mean(axis=0, keepdims=True)
        cents.append(centroid)
        scales.append(jnp.maximum(jnp.max(jnp.abs(pts - centroid)), 1e-6))
    cent = jnp.stack(cents, axis=0)
    scale = jnp.stack(scales, axis=0)[:, None, None]
    pts01 = ((frag_list - cent) / scale * 0.95 + 0.95) / 1.9
    p = pts01.transpose(0, 2, 1).reshape(K, 3, _R, _R)

    scat_idx, scat_val = _run_project(p)

    acc_flat = _make_sc_scatter()(
        scat_idx.reshape(KV * 4 * N // 128, 128),
        scat_val.reshape(2 * KV * 4 * N // 128, 128),
        jnp.zeros((_GR, 128), jnp.float32),
    )
    acc = acc_flat.reshape(KV, 2, 4, _R, 128)

    images_flat, counts_flat = _run_finalize(acc)
    images = images_flat.reshape(K, NV, 2, RES, RES)
    counts = counts_flat.reshape(K, NV, HW)

    pix_corners = scat_idx.reshape(K, NV, 4, N).transpose(0, 3, 1, 2)
    pix_weights = (scat_val.reshape(KV, 2, 4, N)[:, 0]
                   .reshape(K, NV, 4, N).transpose(0, 3, 1, 2))
    return images, pix_corners, pix_weights, counts
